# baseline (speedup 1.0000x reference)
import functools

import jax
import jax.numpy as jnp
import numpy as np
from jax import lax
from jax.experimental import pallas as pl
from jax.experimental.pallas import tpu as pltpu
from jax.experimental.pallas import tpu_sc as plsc

BATCH = 16384
EMB = 64
BN_EPS = 1e-5

NC = 2
NS = 16
NW = NC * NS
B_PER_W = BATCH // NW
CHUNK = 32
NCHUNK = B_PER_W // CHUNK
_D = np.arange(EMB)
_COLPERM = 32 * (_D // 32) + 2 * (_D % 16) + (_D % 32) // 16


def _sc_gather4(user, item, t_ug, t_ig, t_um, t_im):
    mesh = plsc.VectorSubcoreMesh(core_axis_name="c", subcore_axis_name="s")

    out_t = [jax.ShapeDtypeStruct((2 * BATCH, EMB), jnp.bfloat16)] * 4
    scratch = [
        pltpu.VMEM((B_PER_W,), jnp.int32),
        pltpu.VMEM((B_PER_W,), jnp.int32),
    ] + [pltpu.VMEM((2 * CHUNK, EMB), jnp.bfloat16)] * 4 + [
        pltpu.SemaphoreType.DMA,
        pltpu.SemaphoreType.DMA,
    ]

    @functools.partial(pl.kernel, mesh=mesh, out_type=out_t,
                       scratch_types=scratch,
                       compiler_params=pltpu.CompilerParams(
                           needs_layout_passes=False))
    def k(u_hbm, i_hbm, tug, tig, tum, tim,
          o_ug, o_ig, o_um, o_im,
          idx_u, idx_i, bu0, bu1, bi0, bi1, sem0, sem1):
        wid = lax.axis_index("s") * NC + lax.axis_index("c")
        base = wid * B_PER_W
        pltpu.sync_copy(u_hbm.at[pl.ds(base, B_PER_W)], idx_u)
        pltpu.sync_copy(i_hbm.at[pl.ds(base, B_PER_W)], idx_i)

        bu = (bu0, bu1)
        bi = (bi0, bi1)
        sems = (sem0, sem1)

        def fire(tu, ti, c, slot):
            for g in range(CHUNK // 16):
                uu = idx_u[pl.ds(c * CHUNK + g * 16, 16)]
                vv = idx_i[pl.ds(c * CHUNK + g * 16, 16)]
                for l in range(16):
                    j = g * 16 + l
                    pltpu.async_copy(tu.at[pl.ds((uu[l] >> 1) * 2, 2)],
                                     bu[slot].at[pl.ds(2 * j, 2)],
                                     sems[slot])
                    pltpu.async_copy(ti.at[pl.ds((vv[l] >> 1) * 2, 2)],
                                     bi[slot].at[pl.ds(2 * j, 2)],
                                     sems[slot])

        def drain(tu, slot):
            for _ in range(2 * CHUNK):
                pltpu.make_async_copy(tu.at[pl.ds(0, 2)],
                                      bu[slot].at[pl.ds(0, 2)],
                                      sems[slot]).wait()

        def make_pass(tu, ti, ou, oi):
            def body2(t, carry):
                for k2 in range(2):
                    c = 2 * t + k2
                    slot = k2

                    @pl.when(c + 1 < NCHUNK)
                    def _():
                        fire(tu, ti, c + 1, 1 - k2)

                    drain(tu, slot)
                    dst = pl.ds(2 * (base + c * CHUNK), 2 * CHUNK)
                    pltpu.sync_copy(bu[slot], ou.at[dst])
                    pltpu.sync_copy(bi[slot], oi.at[dst])
                return carry

            fire(tu, ti, 0, 0)
            lax.fori_loop(0, NCHUNK // 2, body2, 0)

        make_pass(tug, tig, o_ug, o_ig)
        make_pass(tum, tim, o_um, o_im)

    bf = jnp.bfloat16
    return k(user, item, t_ug.astype(bf), t_ig.astype(bf),
             t_um.astype(bf), t_im.astype(bf))


def _tc_dense(user, item, ugp, igp, ump, imp, wfg, w0a, w0b, b0r,
              w1, b1r, w2, b2r, w3, b3r, wfm, consts):
    BB = 2048
    grid = BATCH // BB

    def sel(pair_ref, h):
        p = pair_ref[...].astype(jnp.float32).reshape(BB, 2, EMB)
        return p[:, 0, :] * (1.0 - h) + p[:, 1, :] * h

    def body(u_r, i_r, ugp_r, igp_r, ump_r, imp_r, wfg_r, w0a_r, w0b_r,
             b0_r, w1_r, b1_r, w2_r, b2_r, w3_r, b3_r, wfm_r, c_r, out_r):
        f32 = jnp.float32
        hu = (u_r[0, :] & 1).astype(f32)[:, None]
        hi = (i_r[0, :] & 1).astype(f32)[:, None]
        ug = sel(ugp_r, hu)
        ig = sel(igp_r, hi)
        um = sel(ump_r, hu)
        im = sel(imp_r, hi)
        x = jnp.dot(um, w0a_r[...], preferred_element_type=f32)
        x = x + jnp.dot(im, w0b_r[...], preferred_element_type=f32)
        x = jnp.maximum(x + b0_r[...], 0.0)
        x = jnp.maximum(jnp.dot(x, w1_r[...], preferred_element_type=f32)
                        + b1_r[...], 0.0)
        x = jnp.maximum(jnp.dot(x, w2_r[...], preferred_element_type=f32)
                        + b2_r[...], 0.0)
        x = jnp.maximum(jnp.dot(x, w3_r[...], preferred_element_type=f32)
                        + b3_r[...], 0.0)
        zg = jnp.sum(ug * ig * wfg_r[...], axis=1)
        zm = jnp.sum(x * wfm_r[...], axis=1)
        z = zg + zm + c_r[0, 0]
        out_r[...] = c_r[0, 1] / (1.0 + jnp.exp(-z)) + c_r[0, 2]

    full = lambda shape: pl.BlockSpec(shape, lambda i: (0, 0))
    pair = lambda: pl.BlockSpec((2 * BB, EMB), lambda i: (i, 0))
    iv = lambda: pl.BlockSpec((1, BB), lambda i: (0, i))
    return pl.pallas_call(
        body,
        grid=(grid,),
        in_specs=[
            iv(), iv(), pair(), pair(), pair(), pair(),
            full((1, EMB)),
            full((EMB, 128)), full((EMB, 128)), full((1, 128)),
            full((128, 128)), full((1, 128)),
            full((128, 128)), full((1, 128)),
            full((128, 128)), full((1, 128)),
            full((1, 128)), full((1, 128)),
        ],
        out_specs=pl.BlockSpec((BB,), lambda i: (i,)),
        out_shape=jax.ShapeDtypeStruct((BATCH,), jnp.float32),
    )(user.reshape(1, BATCH), item.reshape(1, BATCH), ugp, igp, ump, imp,
      wfg, w0a, w0b, b0r, w1, b1r, w2, b2r, w3, b3r, wfm, consts)


def _pad2(a, r, c):
    return jnp.pad(a, ((0, r - a.shape[0]), (0, c - a.shape[1])))


def kernel(user, item, ue_gmf, ie_gmf, ue_mlp, ie_mlp,
           W0, b0, g0, beta0, W1, b1, g1, beta1,
           W2, b2, g2, beta2, W3, b3, g3, beta3,
           Wf, bf, scale, shift):
    ui = user.astype(jnp.int32)
    ii = item.astype(jnp.int32)
    ugp, igp, ump, imp = _sc_gather4(ui, ii, ue_gmf, ie_gmf,
                                     ue_mlp, ie_mlp)

    inv = 1.0 / jnp.sqrt(jnp.float32(1.0 + BN_EPS))

    def fold(W, b, g, beta):
        s = inv * g
        return (W * s[:, None]).T, b * s + beta

    wt0, be0 = fold(W0, b0, g0, beta0)
    w0a, w0b = wt0[:EMB], wt0[EMB:]
    wt1, be1 = fold(W1, b1, g1, beta1)
    wt2, be2 = fold(W2, b2, g2, beta2)
    wt3, be3 = fold(W3, b3, g3, beta3)
    w1 = _pad2(wt1, 128, 128)
    w2 = _pad2(wt2, 128, 128)
    w3 = _pad2(wt3, 128, 128)
    b0r = be0.reshape(1, 128)
    b1r = _pad2(be1.reshape(1, -1), 1, 128)
    b2r = _pad2(be2.reshape(1, -1), 1, 128)
    b3r = _pad2(be3.reshape(1, -1), 1, 128)
    wfg = Wf[:, :EMB]
    wfm = _pad2(Wf[:, EMB:], 1, 128)
    consts = jnp.zeros((1, 128), jnp.float32)
    consts = consts.at[0, 0].set(bf[0]).at[0, 1].set(scale).at[0, 2].set(shift)

    return _tc_dense(ui, ii, ugp, igp, ump, imp, wfg, w0a, w0b, b0r,
                     w1, b1r, w2, b2r, w3, b3r, wfm, consts)


# speedup vs baseline: 1.0361x; 1.0361x over previous
import functools

import jax
import jax.numpy as jnp
import numpy as np
from jax import lax
from jax.experimental import pallas as pl
from jax.experimental.pallas import tpu as pltpu
from jax.experimental.pallas import tpu_sc as plsc

BATCH = 16384
EMB = 64
BN_EPS = 1e-5

NC = 2
NS = 16
NW = NC * NS
B_PER_W = BATCH // NW
CHUNK = 32
NCHUNK = B_PER_W // CHUNK
_D = np.arange(EMB)
_COLPERM = 32 * (_D // 32) + 2 * (_D % 16) + (_D % 32) // 16


def _sc_gather4(user, item, t_ug, t_ig, t_um, t_im):
    mesh = plsc.VectorSubcoreMesh(core_axis_name="c", subcore_axis_name="s")

    out_t = [jax.ShapeDtypeStruct((BATCH, EMB), jnp.float32),
             jax.ShapeDtypeStruct((2 * BATCH, EMB), jnp.bfloat16),
             jax.ShapeDtypeStruct((2 * BATCH, EMB), jnp.bfloat16)]
    scratch = [
        pltpu.VMEM((B_PER_W,), jnp.int32),
        pltpu.VMEM((B_PER_W,), jnp.int32),
    ] + [pltpu.VMEM((CHUNK, EMB), jnp.float32)] * 6 + [
        pltpu.VMEM((2 * CHUNK, EMB), jnp.bfloat16)
    ] * 4 + [
        pltpu.SemaphoreType.DMA,
        pltpu.SemaphoreType.DMA,
    ]

    @functools.partial(pl.kernel, mesh=mesh, out_type=out_t,
                       scratch_types=scratch,
                       compiler_params=pltpu.CompilerParams(
                           needs_layout_passes=False))
    def k(u_hbm, i_hbm, tug, tig, tum, tim,
          o_gmf, o_um, o_im,
          idx_u, idx_i, fu0, fu1, fi0, fi1, p0, p1,
          bu0, bu1, bi0, bi1, sem0, sem1):
        wid = lax.axis_index("s") * NC + lax.axis_index("c")
        base = wid * B_PER_W
        pltpu.sync_copy(u_hbm.at[pl.ds(base, B_PER_W)], idx_u)
        pltpu.sync_copy(i_hbm.at[pl.ds(base, B_PER_W)], idx_i)

        fu = (fu0, fu1)
        fi = (fi0, fi1)
        pb = (p0, p1)
        bu = (bu0, bu1)
        bi = (bi0, bi1)
        sems = (sem0, sem1)

        def fire_f32(c, slot):
            for g in range(CHUNK // 16):
                uu = idx_u[pl.ds(c * CHUNK + g * 16, 16)]
                vv = idx_i[pl.ds(c * CHUNK + g * 16, 16)]
                for l in range(16):
                    j = g * 16 + l
                    pltpu.async_copy(tug.at[pl.ds(uu[l], 1)],
                                     fu[slot].at[pl.ds(j, 1)], sems[slot])
                    pltpu.async_copy(tig.at[pl.ds(vv[l], 1)],
                                     fi[slot].at[pl.ds(j, 1)], sems[slot])

        def drain_f32(slot):
            for _ in range(2 * CHUNK):
                pltpu.make_async_copy(tug.at[pl.ds(0, 1)],
                                      fu[slot].at[pl.ds(0, 1)],
                                      sems[slot]).wait()

        def pass_gmf():
            def body2(t, carry):
                for k2 in range(2):
                    c = 2 * t + k2
                    slot = k2

                    @pl.when(c + 1 < NCHUNK)
                    def _():
                        fire_f32(c + 1, 1 - k2)

                    drain_f32(slot)
                    for l in range(CHUNK):
                        for q in range(EMB // 16):
                            cs = pl.ds(q * 16, 16)
                            pb[slot][l, cs] = (fu[slot][l, cs] *
                                               fi[slot][l, cs])
                    pltpu.sync_copy(
                        pb[slot], o_gmf.at[pl.ds(base + c * CHUNK, CHUNK)])
                return carry

            fire_f32(0, 0)
            lax.fori_loop(0, NCHUNK // 2, body2, 0)

        def fire_bf(c, slot):
            for g in range(CHUNK // 16):
                uu = idx_u[pl.ds(c * CHUNK + g * 16, 16)]
                vv = idx_i[pl.ds(c * CHUNK + g * 16, 16)]
                for l in range(16):
                    j = g * 16 + l
                    pltpu.async_copy(tum.at[pl.ds((uu[l] >> 1) * 2, 2)],
                                     bu[slot].at[pl.ds(2 * j, 2)],
                                     sems[slot])
                    pltpu.async_copy(tim.at[pl.ds((vv[l] >> 1) * 2, 2)],
                                     bi[slot].at[pl.ds(2 * j, 2)],
                                     sems[slot])

        def drain_bf(slot):
            for _ in range(2 * CHUNK):
                pltpu.make_async_copy(tum.at[pl.ds(0, 2)],
                                      bu[slot].at[pl.ds(0, 2)],
                                      sems[slot]).wait()

        def pass_mlp():
            def body2(t, carry):
                for k2 in range(2):
                    c = 2 * t + k2
                    slot = k2

                    @pl.when(c + 1 < NCHUNK)
                    def _():
                        fire_bf(c + 1, 1 - k2)

                    drain_bf(slot)
                    dst = pl.ds(2 * (base + c * CHUNK), 2 * CHUNK)
                    pltpu.sync_copy(bu[slot], o_um.at[dst])
                    pltpu.sync_copy(bi[slot], o_im.at[dst])
                return carry

            fire_bf(0, 0)
            lax.fori_loop(0, NCHUNK // 2, body2, 0)

        pass_gmf()
        pass_mlp()

    bf = jnp.bfloat16
    return k(user, item, t_ug, t_ig, t_um.astype(bf), t_im.astype(bf))


def _tc_dense(user, item, gmf, ump, imp, wfg, w0a, w0b, b0r,
              w1, b1r, w2, b2r, w3, b3r, wfm, consts):
    BB = 2048
    grid = BATCH // BB

    def sel(pair_ref, h):
        p = pair_ref[...].astype(jnp.float32).reshape(BB, 2, EMB)
        return p[:, 0, :] * (1.0 - h) + p[:, 1, :] * h

    def body(u_r, i_r, gmf_r, ump_r, imp_r, wfg_r, w0a_r, w0b_r,
             b0_r, w1_r, b1_r, w2_r, b2_r, w3_r, b3_r, wfm_r, c_r, out_r):
        f32 = jnp.float32
        hu = (u_r[0, :] & 1).astype(f32)[:, None]
        hi = (i_r[0, :] & 1).astype(f32)[:, None]
        um = sel(ump_r, hu)
        im = sel(imp_r, hi)
        x = jnp.dot(um, w0a_r[...], preferred_element_type=f32)
        x = x + jnp.dot(im, w0b_r[...], preferred_element_type=f32)
        x = jnp.maximum(x + b0_r[...], 0.0)
        x = jnp.maximum(jnp.dot(x, w1_r[...], preferred_element_type=f32)
                        + b1_r[...], 0.0)
        x = jnp.maximum(jnp.dot(x, w2_r[...], preferred_element_type=f32)
                        + b2_r[...], 0.0)
        x = jnp.maximum(jnp.dot(x, w3_r[...], preferred_element_type=f32)
                        + b3_r[...], 0.0)
        zg = jnp.sum(gmf_r[...] * wfg_r[...], axis=1)
        zm = jnp.sum(x * wfm_r[...], axis=1)
        z = zg + zm + c_r[0, 0]
        out_r[...] = c_r[0, 1] / (1.0 + jnp.exp(-z)) + c_r[0, 2]

    full = lambda shape: pl.BlockSpec(shape, lambda i: (0, 0))
    pair = lambda: pl.BlockSpec((2 * BB, EMB), lambda i: (i, 0))
    iv = lambda: pl.BlockSpec((1, BB), lambda i: (0, i))
    return pl.pallas_call(
        body,
        grid=(grid,),
        in_specs=[
            iv(), iv(), pl.BlockSpec((BB, EMB), lambda i: (i, 0)),
            pair(), pair(),
            full((1, EMB)),
            full((EMB, 128)), full((EMB, 128)), full((1, 128)),
            full((128, 128)), full((1, 128)),
            full((128, 128)), full((1, 128)),
            full((128, 128)), full((1, 128)),
            full((1, 128)), full((1, 128)),
        ],
        out_specs=pl.BlockSpec((BB,), lambda i: (i,)),
        out_shape=jax.ShapeDtypeStruct((BATCH,), jnp.float32),
    )(user.reshape(1, BATCH), item.reshape(1, BATCH), gmf, ump, imp,
      wfg, w0a, w0b, b0r, w1, b1r, w2, b2r, w3, b3r, wfm, consts)


def _pad2(a, r, c):
    return jnp.pad(a, ((0, r - a.shape[0]), (0, c - a.shape[1])))


def kernel(user, item, ue_gmf, ie_gmf, ue_mlp, ie_mlp,
           W0, b0, g0, beta0, W1, b1, g1, beta1,
           W2, b2, g2, beta2, W3, b3, g3, beta3,
           Wf, bf, scale, shift):
    ui = user.astype(jnp.int32)
    ii = item.astype(jnp.int32)
    gmf, ump, imp = _sc_gather4(ui, ii, ue_gmf, ie_gmf,
                                ue_mlp, ie_mlp)

    inv = 1.0 / jnp.sqrt(jnp.float32(1.0 + BN_EPS))

    def fold(W, b, g, beta):
        s = inv * g
        return (W * s[:, None]).T, b * s + beta

    wt0, be0 = fold(W0, b0, g0, beta0)
    w0a, w0b = wt0[:EMB], wt0[EMB:]
    wt1, be1 = fold(W1, b1, g1, beta1)
    wt2, be2 = fold(W2, b2, g2, beta2)
    wt3, be3 = fold(W3, b3, g3, beta3)
    w1 = _pad2(wt1, 128, 128)
    w2 = _pad2(wt2, 128, 128)
    w3 = _pad2(wt3, 128, 128)
    b0r = be0.reshape(1, 128)
    b1r = _pad2(be1.reshape(1, -1), 1, 128)
    b2r = _pad2(be2.reshape(1, -1), 1, 128)
    b3r = _pad2(be3.reshape(1, -1), 1, 128)
    wfg = Wf[:, :EMB]
    wfm = _pad2(Wf[:, EMB:], 1, 128)
    consts = jnp.zeros((1, 128), jnp.float32)
    consts = consts.at[0, 0].set(bf[0]).at[0, 1].set(scale).at[0, 2].set(shift)

    return _tc_dense(ui, ii, gmf, ump, imp, wfg, w0a, w0b, b0r,
                     w1, b1r, w2, b2r, w3, b3r, wfm, consts)


# speedup vs baseline: 1.4503x; 1.3997x over previous
import functools

import jax
import jax.numpy as jnp
import numpy as np
from jax import lax
from jax.experimental import pallas as pl
from jax.experimental.pallas import tpu as pltpu
from jax.experimental.pallas import tpu_sc as plsc

BATCH = 16384
EMB = 64
BN_EPS = 1e-5

NC = 2
NS = 16
NW = NC * NS
B_PER_W = BATCH // NW
CHUNK = 32
NCHUNK = B_PER_W // CHUNK
_D = np.arange(EMB)
_COLPERM = 32 * (_D // 32) + 2 * (_D % 16) + (_D % 32) // 16


def _sc_gather4(user, item, t_ug, t_ig, t_um, t_im):
    mesh = plsc.VectorSubcoreMesh(core_axis_name="c", subcore_axis_name="s")

    out_t = [jax.ShapeDtypeStruct((BATCH, EMB), jnp.float32),
             jax.ShapeDtypeStruct((2 * BATCH, EMB), jnp.bfloat16),
             jax.ShapeDtypeStruct((2 * BATCH, EMB), jnp.bfloat16)]
    scratch = [
        pltpu.VMEM((B_PER_W,), jnp.int32),
        pltpu.VMEM((B_PER_W,), jnp.int32),
    ] + [pltpu.VMEM((CHUNK, EMB), jnp.float32)] * 6 + [
        pltpu.VMEM((2 * CHUNK, EMB), jnp.bfloat16)
    ] * 4 + [
        pltpu.SemaphoreType.DMA,
        pltpu.SemaphoreType.DMA,
    ]

    @functools.partial(pl.kernel, mesh=mesh, out_type=out_t,
                       scratch_types=scratch,
                       compiler_params=pltpu.CompilerParams(
                           needs_layout_passes=False))
    def k(u_hbm, i_hbm, tug, tig, tum, tim,
          o_gmf, o_um, o_im,
          idx_u, idx_i, fu0, fu1, fi0, fi1, p0, p1,
          bu0, bu1, bi0, bi1, sem0, sem1):
        wid = lax.axis_index("s") * NC + lax.axis_index("c")
        base = wid * B_PER_W
        pltpu.sync_copy(u_hbm.at[pl.ds(base, B_PER_W)], idx_u)
        pltpu.sync_copy(i_hbm.at[pl.ds(base, B_PER_W)], idx_i)

        fu = (fu0, fu1)
        fi = (fi0, fi1)
        pb = (p0, p1)
        bu = (bu0, bu1)
        bi = (bi0, bi1)
        sems = (sem0, sem1)

        def fire_f32(c, slot):
            for g in range(CHUNK // 16):
                uu = idx_u[pl.ds(c * CHUNK + g * 16, 16)]
                vv = idx_i[pl.ds(c * CHUNK + g * 16, 16)]
                for l in range(16):
                    j = g * 16 + l
                    pltpu.async_copy(tug.at[uu[l] >> 4, uu[l] & 15],
                                     fu[slot].at[j], sems[slot])
                    pltpu.async_copy(tig.at[vv[l] >> 4, vv[l] & 15],
                                     fi[slot].at[j], sems[slot])

        def drain_f32(slot):
            for _ in range(2 * CHUNK):
                pltpu.make_async_copy(tug.at[0, 0],
                                      fu[slot].at[0],
                                      sems[slot]).wait()

        def pass_gmf():
            def body2(t, carry):
                for k2 in range(2):
                    c = 2 * t + k2
                    slot = k2

                    @pl.when(c + 1 < NCHUNK)
                    def _():
                        fire_f32(c + 1, 1 - k2)

                    drain_f32(slot)
                    for l in range(CHUNK):
                        for q in range(EMB // 16):
                            cs = pl.ds(q * 16, 16)
                            pb[slot][l, cs] = (fu[slot][l, cs] *
                                               fi[slot][l, cs])
                    pltpu.sync_copy(
                        pb[slot], o_gmf.at[pl.ds(base + c * CHUNK, CHUNK)])
                return carry

            fire_f32(0, 0)
            lax.fori_loop(0, NCHUNK // 2, body2, 0)

        def fire_bf(c, slot):
            for g in range(CHUNK // 16):
                uu = idx_u[pl.ds(c * CHUNK + g * 16, 16)]
                vv = idx_i[pl.ds(c * CHUNK + g * 16, 16)]
                for l in range(16):
                    j = g * 16 + l
                    pltpu.async_copy(tum.at[pl.ds((uu[l] >> 1) * 2, 2)],
                                     bu[slot].at[pl.ds(2 * j, 2)],
                                     sems[slot])
                    pltpu.async_copy(tim.at[pl.ds((vv[l] >> 1) * 2, 2)],
                                     bi[slot].at[pl.ds(2 * j, 2)],
                                     sems[slot])

        def drain_bf(slot):
            for _ in range(2 * CHUNK):
                pltpu.make_async_copy(tum.at[pl.ds(0, 2)],
                                      bu[slot].at[pl.ds(0, 2)],
                                      sems[slot]).wait()

        def pass_mlp():
            def body2(t, carry):
                for k2 in range(2):
                    c = 2 * t + k2
                    slot = k2

                    @pl.when(c + 1 < NCHUNK)
                    def _():
                        fire_bf(c + 1, 1 - k2)

                    drain_bf(slot)
                    dst = pl.ds(2 * (base + c * CHUNK), 2 * CHUNK)
                    pltpu.sync_copy(bu[slot], o_um.at[dst])
                    pltpu.sync_copy(bi[slot], o_im.at[dst])
                return carry

            fire_bf(0, 0)
            lax.fori_loop(0, NCHUNK // 2, body2, 0)

        pass_gmf()
        pass_mlp()

    bf = jnp.bfloat16
    return k(user, item,
             t_ug.reshape(62500, 16, EMB), t_ig.reshape(62500, 16, EMB),
             t_um.astype(bf), t_im.astype(bf))


def _tc_dense(user, item, gmf, ump, imp, wfg, w0a, w0b, b0r,
              w1, b1r, w2, b2r, w3, b3r, wfm, consts):
    BB = 2048
    grid = BATCH // BB

    def sel(pair_ref, h):
        p = pair_ref[...].astype(jnp.float32).reshape(BB, 2, EMB)
        return p[:, 0, :] * (1.0 - h) + p[:, 1, :] * h

    def body(u_r, i_r, gmf_r, ump_r, imp_r, wfg_r, w0a_r, w0b_r,
             b0_r, w1_r, b1_r, w2_r, b2_r, w3_r, b3_r, wfm_r, c_r, out_r):
        f32 = jnp.float32
        hu = (u_r[0, :] & 1).astype(f32)[:, None]
        hi = (i_r[0, :] & 1).astype(f32)[:, None]
        um = sel(ump_r, hu)
        im = sel(imp_r, hi)
        x = jnp.dot(um, w0a_r[...], preferred_element_type=f32)
        x = x + jnp.dot(im, w0b_r[...], preferred_element_type=f32)
        x = jnp.maximum(x + b0_r[...], 0.0)
        x = jnp.maximum(jnp.dot(x, w1_r[...], preferred_element_type=f32)
                        + b1_r[...], 0.0)
        x = jnp.maximum(jnp.dot(x, w2_r[...], preferred_element_type=f32)
                        + b2_r[...], 0.0)
        x = jnp.maximum(jnp.dot(x, w3_r[...], preferred_element_type=f32)
                        + b3_r[...], 0.0)
        zg = jnp.sum(gmf_r[...] * wfg_r[...], axis=1)
        zm = jnp.sum(x * wfm_r[...], axis=1)
        z = zg + zm + c_r[0, 0]
        out_r[...] = c_r[0, 1] / (1.0 + jnp.exp(-z)) + c_r[0, 2]

    full = lambda shape: pl.BlockSpec(shape, lambda i: (0, 0))
    pair = lambda: pl.BlockSpec((2 * BB, EMB), lambda i: (i, 0))
    iv = lambda: pl.BlockSpec((1, BB), lambda i: (0, i))
    return pl.pallas_call(
        body,
        grid=(grid,),
        in_specs=[
            iv(), iv(), pl.BlockSpec((BB, EMB), lambda i: (i, 0)),
            pair(), pair(),
            full((1, EMB)),
            full((EMB, 128)), full((EMB, 128)), full((1, 128)),
            full((128, 128)), full((1, 128)),
            full((128, 128)), full((1, 128)),
            full((128, 128)), full((1, 128)),
            full((1, 128)), full((1, 128)),
        ],
        out_specs=pl.BlockSpec((BB,), lambda i: (i,)),
        out_shape=jax.ShapeDtypeStruct((BATCH,), jnp.float32),
    )(user.reshape(1, BATCH), item.reshape(1, BATCH), gmf, ump, imp,
      wfg, w0a, w0b, b0r, w1, b1r, w2, b2r, w3, b3r, wfm, consts)


def _pad2(a, r, c):
    return jnp.pad(a, ((0, r - a.shape[0]), (0, c - a.shape[1])))


def kernel(user, item, ue_gmf, ie_gmf, ue_mlp, ie_mlp,
           W0, b0, g0, beta0, W1, b1, g1, beta1,
           W2, b2, g2, beta2, W3, b3, g3, beta3,
           Wf, bf, scale, shift):
    ui = user.astype(jnp.int32)
    ii = item.astype(jnp.int32)
    gmf, ump, imp = _sc_gather4(ui, ii, ue_gmf, ie_gmf,
                                ue_mlp, ie_mlp)

    inv = 1.0 / jnp.sqrt(jnp.float32(1.0 + BN_EPS))

    def fold(W, b, g, beta):
        s = inv * g
        return (W * s[:, None]).T, b * s + beta

    wt0, be0 = fold(W0, b0, g0, beta0)
    w0a, w0b = wt0[:EMB], wt0[EMB:]
    wt1, be1 = fold(W1, b1, g1, beta1)
    wt2, be2 = fold(W2, b2, g2, beta2)
    wt3, be3 = fold(W3, b3, g3, beta3)
    w1 = _pad2(wt1, 128, 128)
    w2 = _pad2(wt2, 128, 128)
    w3 = _pad2(wt3, 128, 128)
    b0r = be0.reshape(1, 128)
    b1r = _pad2(be1.reshape(1, -1), 1, 128)
    b2r = _pad2(be2.reshape(1, -1), 1, 128)
    b3r = _pad2(be3.reshape(1, -1), 1, 128)
    wfg = Wf[:, :EMB]
    wfm = _pad2(Wf[:, EMB:], 1, 128)
    consts = jnp.zeros((1, 128), jnp.float32)
    consts = consts.at[0, 0].set(bf[0]).at[0, 1].set(scale).at[0, 2].set(shift)

    return _tc_dense(ui, ii, gmf, ump, imp, wfg, w0a, w0b, b0r,
                     w1, b1r, w2, b2r, w3, b3r, wfm, consts)
